# Initial kernel scaffold; baseline (speedup 1.0000x reference)
import jax
import jax.numpy as jnp
from jax.experimental import pallas as pl


def kernel(z_h, z_l, edge_index_h_l, We1, be1, We2, be2, Ww1, bw1, Ww2, bw2, Wn1, bn1, Wn2, bn2):
    raise NotImplementedError("write your pallas kernel here")


# speedup vs baseline: 14.1204x; 14.1204x over previous
import functools

import jax
import jax.numpy as jnp
from jax import lax
from jax.experimental import pallas as pl
from jax.experimental.pallas import tpu as pltpu
from jax.experimental.pallas import tpu_sc as plsc

F = 13
MSG = 16

_SC_TILES = 32
_GW = 128


def _sc_mesh():
    return plsc.VectorSubcoreMesh(core_axis_name="c", subcore_axis_name="s")


_SC_PARAMS = pltpu.CompilerParams(use_tc_tiling_on_sc=False)


_FP = 16


def _sc_gather(zh, zl, src2, tgt2):
    E = src2.shape[1]

    @functools.partial(
        pl.kernel,
        out_type=(
            jax.ShapeDtypeStruct((E, _FP), jnp.float32),
            jax.ShapeDtypeStruct((E, _FP), jnp.float32),
        ),
        mesh=_sc_mesh(),
        compiler_params=_SC_PARAMS,
    )
    def gk(zh_hbm, zl_hbm, src_hbm, tgt_hbm, zs_hbm, zt_hbm):
        def body(src_v, tgt_v, zs_v, zt_v):
            pltpu.sync_copy(zh_hbm.at[src_v.at[0]], zs_v)
            pltpu.sync_copy(zl_hbm.at[tgt_v.at[0]], zt_v)

        pltpu.emit_pipeline(
            body,
            grid=(E // _GW,),
            in_specs=[
                pl.BlockSpec((1, _GW), lambda i: (0, i)),
                pl.BlockSpec((1, _GW), lambda i: (0, i)),
            ],
            out_specs=[
                pl.BlockSpec((_GW, _FP), lambda i: (i, 0)),
                pl.BlockSpec((_GW, _FP), lambda i: (i, 0)),
            ],
            core_axis_name=("c", "s"),
            dimension_semantics=(pltpu.PARALLEL,),
        )(src_hbm, tgt_hbm, zs_hbm, zt_hbm)

    return gk(zh, zl, src2, tgt2)


def _sc_scatter_add(wm, tgt2, n_nodes):
    E = tgt2.shape[1]
    rows_per_tile = n_nodes // 16
    zr = 1250
    n_zero_copies = rows_per_tile // zr

    @functools.partial(
        pl.kernel,
        out_type=jax.ShapeDtypeStruct((2, n_nodes, MSG), jnp.float32),
        mesh=_sc_mesh(),
        compiler_params=_SC_PARAMS,
        scratch_types=[
            pltpu.VMEM_SHARED((n_nodes, MSG), jnp.float32),
            pltpu.VMEM((zr, MSG), jnp.float32),
        ],
    )
    def sk(wm_hbm, tgt_hbm, o_hbm, acc, zbuf):
        c = lax.axis_index("c")
        s = lax.axis_index("s")

        @pl.loop(0, zr)
        def _(r):
            zbuf[r, :] = jnp.zeros((MSG,), jnp.float32)

        @pl.loop(0, n_zero_copies)
        def _(k):
            pltpu.sync_copy(zbuf, acc.at[pl.ds(s * rows_per_tile + k * zr, zr)])

        plsc.subcore_barrier()

        def body(wm_v, tgt_v):
            pltpu.sync_copy(wm_v, acc.at[tgt_v.at[0]], add=True)

        pltpu.emit_pipeline(
            body,
            grid=(E // _GW,),
            in_specs=[
                pl.BlockSpec((_GW, MSG), lambda i: (i, 0)),
                pl.BlockSpec((1, _GW), lambda i: (0, i)),
            ],
            out_specs=[],
            core_axis_name=("c", "s"),
            dimension_semantics=(pltpu.PARALLEL,),
        )(wm_hbm, tgt_hbm)

        plsc.subcore_barrier()
        pltpu.sync_copy(
            acc.at[pl.ds(s * rows_per_tile, rows_per_tile)],
            o_hbm.at[c, pl.ds(s * rows_per_tile, rows_per_tile)],
        )

    return sk(wm, tgt2)


def _edge_mlp_body(zs_ref, zt_ref, w1t_ref, b1_ref, w2t_ref, b2_ref, out_ref):
    zst = zs_ref[...].T
    ztt = zt_ref[...].T
    diff = zst[0:3, :] - ztt[0:3, :]
    dist = jnp.sum(diff * diff, axis=0, keepdims=True)
    a1, a2, a3 = zst[3:4, :], zst[4:5, :], zst[5:6, :]
    b1, b2, b3 = ztt[3:4, :], ztt[4:5, :], ztt[5:6, :]
    c1 = a2 * b3 - a3 * b2
    c2 = a3 * b1 - a1 * b3
    c3 = a1 * b2 - a2 * b1
    abscp = jnp.sqrt(c1 * c1 + c2 * c2 + c3 * c3)
    inp = jnp.concatenate([zst, ztt, diff, dist, c1, c2, c3, abscp], axis=0)
    h = jnp.tanh(
        jnp.dot(w1t_ref[...], inp, preferred_element_type=jnp.float32)
        + b1_ref[...]
    )
    o = (
        jnp.dot(w2t_ref[...], h, preferred_element_type=jnp.float32)
        + b2_ref[...]
    )
    wmt = o[0:MSG, :] * jax.nn.sigmoid(o[MSG:MSG + 1, :])
    out_ref[...] = wmt.T


def _tc_edge_mlp(zs, zt, w1t, b1, w2t, b2):
    E = zs.shape[0]
    T = 6400
    in_edge = w1t.shape[1]
    hid2 = w1t.shape[0]
    return pl.pallas_call(
        _edge_mlp_body,
        grid=(E // T,),
        in_specs=[
            pl.BlockSpec((T, _FP), lambda i: (i, 0)),
            pl.BlockSpec((T, _FP), lambda i: (i, 0)),
            pl.BlockSpec((hid2, in_edge), lambda i: (0, 0)),
            pl.BlockSpec((hid2, 1), lambda i: (0, 0)),
            pl.BlockSpec((MSG + 1, hid2), lambda i: (0, 0)),
            pl.BlockSpec((MSG + 1, 1), lambda i: (0, 0)),
        ],
        out_specs=pl.BlockSpec((T, MSG), lambda i: (i, 0)),
        out_shape=jax.ShapeDtypeStruct((E, MSG), jnp.float32),
    )(zs, zt, w1t, b1, w2t, b2)


def _node_mlp_body(zl_ref, p0_ref, p1_ref, wn1_ref, bn1_ref, wn2_ref, bn2_ref,
                   out_ref):
    magg = p0_ref[...] + p1_ref[...]
    inp = jnp.concatenate([zl_ref[...], magg], axis=-1)
    h = jnp.tanh(
        jnp.dot(inp, wn1_ref[...], preferred_element_type=jnp.float32)
        + bn1_ref[...]
    )
    out_ref[...] = (
        jnp.dot(h, wn2_ref[...], preferred_element_type=jnp.float32)
        + bn2_ref[...]
    )


def _tc_node_mlp(zl, p0, p1, wn1, bn1, wn2, bn2):
    n = zl.shape[0]
    T = 4000
    hid = wn1.shape[1]
    return pl.pallas_call(
        _node_mlp_body,
        grid=(n // T,),
        in_specs=[
            pl.BlockSpec((T, F), lambda i: (i, 0)),
            pl.BlockSpec((T, MSG), lambda i: (i, 0)),
            pl.BlockSpec((T, MSG), lambda i: (i, 0)),
            pl.BlockSpec((F + MSG, hid), lambda i: (0, 0)),
            pl.BlockSpec((1, hid), lambda i: (0, 0)),
            pl.BlockSpec((hid, F), lambda i: (0, 0)),
            pl.BlockSpec((1, F), lambda i: (0, 0)),
        ],
        out_specs=pl.BlockSpec((T, F), lambda i: (i, 0)),
        out_shape=jax.ShapeDtypeStruct((n, F), jnp.float32),
    )(zl, p0, p1, wn1, bn1, wn2, bn2)


def kernel(z_h, z_l, edge_index_h_l, We1, be1, We2, be2, Ww1, bw1, Ww2, bw2,
           Wn1, bn1, Wn2, bn2):
    B, n_h, _ = z_h.shape
    n_l = z_l.shape[1]
    E = edge_index_h_l.shape[2]
    hid = We1.shape[1]

    w1c = jnp.concatenate([We1, Ww1], axis=1)
    pad = jnp.zeros((_FP - F, 2 * hid), jnp.float32)
    w1 = jnp.concatenate([w1c[:F], pad, w1c[F:2 * F], pad, w1c[2 * F:]], axis=0)
    w1t = w1.T
    b1 = jnp.concatenate([be1, bw1]).reshape(-1, 1)
    w2 = jnp.zeros((2 * hid, MSG + 1), jnp.float32)
    w2 = w2.at[:hid, :MSG].set(We2).at[hid:, MSG:].set(Ww2)
    w2t = w2.T
    b2 = jnp.concatenate([be2, bw2]).reshape(-1, 1)
    bn1r = bn1.reshape(1, -1)
    bn2r = bn2.reshape(1, -1)

    zpad = ((0, 0), (0, _FP - F))
    outs = []
    for b in range(B):
        src2 = edge_index_h_l[b, 0].reshape(1, E)
        tgt2 = edge_index_h_l[b, 1].reshape(1, E)
        zh_p = jnp.pad(z_h[b], zpad)
        zl_p = jnp.pad(z_l[b], zpad)
        zs, zt = _sc_gather(zh_p, zl_p, src2, tgt2)
        wm = _tc_edge_mlp(zs, zt, w1t, b1, w2t, b2)
        partials = _sc_scatter_add(wm, tgt2, n_l)
        delta = _tc_node_mlp(z_l[b], partials[0], partials[1],
                             Wn1, bn1r, Wn2, bn2r)
        outs.append(delta)
    return jnp.stack(outs, axis=0)
